# Initial kernel scaffold; baseline (speedup 1.0000x reference)
import jax
import jax.numpy as jnp
from jax.experimental import pallas as pl


def kernel(output, xt):
    raise NotImplementedError("write your pallas kernel here")


# speedup vs baseline: 1.4036x; 1.4036x over previous
import jax
import jax.numpy as jnp
from jax.experimental import pallas as pl
from jax.experimental.pallas import tpu as pltpu

_B, _L, _V = 32, 32, 32001
_MASK = 32000
_ROWS = _B * _L
_BLK_R = 8


def _row_kernel(x_ref, xt_ref, o_ref):
    x = x_ref[...]
    xt = xt_ref[...]
    col = jax.lax.broadcasted_iota(jnp.int32, x.shape, 1)
    neg_inf = jnp.float32(-jnp.inf)
    xm = jnp.where((col == _MASK) | (col >= _V), neg_inf, x)
    m = jnp.max(xm, axis=1, keepdims=True)
    s = jnp.sum(jnp.exp(xm - m), axis=1, keepdims=True)
    logsm = xm - (m + jnp.log(s))
    onehot_row = jnp.where(col == xt, jnp.float32(0.0), neg_inf)
    o_ref[...] = jnp.where(xt != _MASK, onehot_row, logsm)


def kernel(output, xt):
    x2d = output.reshape(_ROWS, _V)
    xt2d = xt.reshape(_ROWS, 1)
    out = pl.pallas_call(
        _row_kernel,
        grid=(_ROWS // _BLK_R,),
        in_specs=[
            pl.BlockSpec((_BLK_R, _V), lambda i: (i, 0)),
            pl.BlockSpec((_BLK_R, 1), lambda i: (i, 0)),
        ],
        out_specs=pl.BlockSpec((_BLK_R, _V), lambda i: (i, 0)),
        out_shape=jax.ShapeDtypeStruct((_ROWS, _V), jnp.float32),
    )(x2d, xt2d)
    return out.reshape(_B, _L, _V)
